# baseline (speedup 1.0000x reference)
import functools

import jax
import jax.numpy as jnp
from jax import lax
from jax.experimental import pallas as pl
from jax.experimental.pallas import tpu as pltpu
from jax.experimental.pallas import tpu_sc as plsc

N_GENE, N_CELL, N_DRUG = 8000, 1500, 500
N = N_GENE + N_CELL + N_DRUG
D = 128
F2 = D // 2
E = 320000

NC, NS = 2, 16
NW = NC * NS
CHUNK = 128
CPT = 80
CPT2 = 160
EP = NW * CPT * CHUNK
NPAD = 10112
RPT = NPAD // NS
NBUF = 4

_f32 = jnp.float32


def _fold_body(wg_ref, wc_ref, wd_ref, wcv_ref, bs_ref,
               wg2_ref, wc2_ref, wd2_ref, bb_ref):
    wcv = wcv_ref[...]
    wg2_ref[...] = jnp.dot(wg_ref[...], wcv, preferred_element_type=_f32)
    wc2_ref[...] = jnp.dot(wc_ref[...], wcv, preferred_element_type=_f32)
    wd2_ref[...] = jnp.dot(wd_ref[...], wcv, preferred_element_type=_f32)
    bb_ref[...] = jnp.dot(bs_ref[...], wcv, preferred_element_type=_f32)


def _fold(wg, wc, wd, wcv, bstack):
    return pl.pallas_call(
        _fold_body,
        out_shape=[
            jax.ShapeDtypeStruct((512, D), _f32),
            jax.ShapeDtypeStruct((256, D), _f32),
            jax.ShapeDtypeStruct((128, D), _f32),
            jax.ShapeDtypeStruct((3, D), _f32),
        ],
    )(wg, wc, wd, wcv, bstack)


def _proj_body(x_ref, w_ref, bb_ref, o_ref):
    o_ref[...] = jnp.dot(x_ref[...], w_ref[...],
                         preferred_element_type=_f32) + bb_ref[...]


def _proj_grid(x, w2, bb_row, blk):
    m, k = x.shape
    return pl.pallas_call(
        _proj_body,
        grid=(m // blk,),
        in_specs=[
            pl.BlockSpec((blk, k), lambda i: (i, 0)),
            pl.BlockSpec((k, D), lambda i: (0, 0)),
            pl.BlockSpec((1, D), lambda i: (0, 0)),
        ],
        out_specs=pl.BlockSpec((blk, D), lambda i: (i, 0)),
        out_shape=jax.ShapeDtypeStruct((m, D), _f32),
    )(x, w2, bb_row)


def _proj_whole(x, w2, bb_row):
    m = x.shape[0]
    return pl.pallas_call(
        _proj_body,
        out_shape=jax.ShapeDtypeStruct((m, D), _f32),
    )(x, w2, bb_row)


def _scale_body(h_ref, hist_ref, y2_ref):
    deg = 1.0 + hist_ref[0, :, 0:1] + hist_ref[1, :, 0:1]
    dinv = lax.rsqrt(deg)
    y2_ref[...] = h_ref[...] * dinv


def _scale(h, hist3):
    return pl.pallas_call(
        _scale_body,
        out_shape=jax.ShapeDtypeStruct((NPAD, D), _f32),
    )(h, hist3)


def _final_body(acc_ref, y_ref, hist_ref, b_ref, o_ref):
    deg = 1.0 + hist_ref[0, :, 0:1] + hist_ref[1, :, 0:1]
    dinv = lax.rsqrt(deg)
    acc = acc_ref[...]
    agg = jnp.concatenate([acc[:N], acc[NPAD:NPAD + N]], axis=1)
    o_ref[...] = (agg + y_ref[0:N]) * dinv[:N] + b_ref[...]


def _final(acc, y, hist3, brow):
    return pl.pallas_call(
        _final_body,
        out_shape=jax.ShapeDtypeStruct((N, D), _f32),
    )(acc, y, hist3, brow)


_sc_mesh = plsc.VectorSubcoreMesh(
    core_axis_name="c", subcore_axis_name="s", num_cores=NC, num_subcores=NS)
_sc_params = pltpu.CompilerParams(use_tc_tiling_on_sc=False)


@functools.partial(
    pl.kernel,
    out_type=jax.ShapeDtypeStruct((NC * NPAD, 16), _f32),
    mesh=_sc_mesh,
    scratch_types=[
        pltpu.VMEM((CPT, CHUNK), jnp.int32),
        pltpu.VMEM((CHUNK, 16), _f32),
        pltpu.SemaphoreType.DMA,
        pltpu.VMEM_SHARED((NPAD, 16), _f32),
    ],
    compiler_params=_sc_params,
)
def _deg(colp_ref, ones_ref, z16_ref, out_ref, colbuf, ones_v, sem_sc, hist_sh):
    cid = lax.axis_index("c")
    sid = lax.axis_index("s")
    wid = sid * NC + cid
    pltpu.sync_copy(z16_ref, hist_sh.at[pl.ds(sid * RPT, RPT)])
    pltpu.sync_copy(ones_ref, ones_v)
    pltpu.sync_copy(colp_ref.at[wid], colbuf)
    plsc.subcore_barrier()

    def fire(j, carry):
        pltpu.make_async_copy(
            ones_v, hist_sh.at[colbuf.at[j]], sem_sc).start(add=True)
        return carry

    lax.fori_loop(0, CPT, fire, 0)

    def drain(j, carry):
        pltpu.make_async_copy(
            ones_v, hist_sh.at[colbuf.at[0]], sem_sc).wait()
        return carry

    lax.fori_loop(0, CPT, drain, 0)
    plsc.subcore_barrier()
    pltpu.sync_copy(hist_sh.at[pl.ds(sid * RPT, RPT)],
                    out_ref.at[pl.ds(cid * NPAD + sid * RPT, RPT)])


@functools.partial(
    pl.kernel,
    out_type=jax.ShapeDtypeStruct((NC * NPAD, F2), _f32),
    mesh=_sc_mesh,
    scratch_types=[
        pltpu.VMEM((CPT2, CHUNK), jnp.int32),
        pltpu.VMEM((CPT2, CHUNK), jnp.int32),
        pltpu.VMEM((NBUF, CHUNK, F2), _f32),
        pltpu.SemaphoreType.DMA,
        pltpu.SemaphoreType.DMA,
        pltpu.SemaphoreType.DMA,
        pltpu.SemaphoreType.DMA,
        pltpu.SemaphoreType.DMA,
        pltpu.SemaphoreType.DMA,
        pltpu.SemaphoreType.DMA,
        pltpu.SemaphoreType.DMA,
        pltpu.VMEM_SHARED((NPAD, F2), _f32),
    ],
    compiler_params=_sc_params,
)
def _edge(y2_ref, rowp_ref, colp_ref, z64_ref, out_ref,
          rowbuf, colbuf, rows, g0, g1, g2, g3, s0, s1, s2, s3, acc_sh):
    gsems = (g0, g1, g2, g3)
    ssems = (s0, s1, s2, s3)
    cid = lax.axis_index("c")
    sid = lax.axis_index("s")
    pltpu.sync_copy(z64_ref, acc_sh.at[pl.ds(sid * RPT, RPT)])
    pltpu.sync_copy(rowp_ref.at[cid * NS + sid], rowbuf)
    pltpu.sync_copy(colp_ref.at[sid], colbuf)
    plsc.subcore_barrier()

    for b in range(NBUF):
        pltpu.make_async_copy(
            y2_ref.at[rowbuf.at[b]], rows.at[b], gsems[b]).start()

    ngroups = CPT2 // NBUF

    def group(g, carry):
        for b in range(NBUF):
            j = g * NBUF + b
            pltpu.make_async_copy(
                y2_ref.at[rowbuf.at[b]], rows.at[b], gsems[b]).wait()
            pltpu.make_async_copy(
                rows.at[b], acc_sh.at[colbuf.at[j]], ssems[b]).start(add=True)
            pltpu.make_async_copy(
                rows.at[b], acc_sh.at[colbuf.at[0]], ssems[b]).wait()

            @pl.when(g < ngroups - 1)
            def _():
                pltpu.make_async_copy(
                    y2_ref.at[rowbuf.at[j + NBUF]], rows.at[b],
                    gsems[b]).start()
        return carry

    lax.fori_loop(0, ngroups, group, 0)
    plsc.subcore_barrier()
    pltpu.sync_copy(acc_sh.at[pl.ds(sid * RPT, RPT)],
                    out_ref.at[pl.ds(cid * NPAD + sid * RPT, RPT)])


def kernel(raw_gene_feats, raw_cell_feats, raw_drug_feats,
           gene_idx, cell_idx, drug_idx, edge_index,
           W_gene, b_gene, W_cell, b_cell, W_drug, b_drug, W_conv, b_conv):
    del gene_idx, cell_idx, drug_idx

    bstack = jnp.stack([b_gene, b_cell, b_drug])
    wg2, wc2, wd2, bb = _fold(W_gene, W_cell, W_drug, W_conv, bstack)

    hg = _proj_grid(raw_gene_feats, wg2, bb[0:1], 1000)
    hc = _proj_whole(raw_cell_feats, wc2, bb[1:2])
    hd = _proj_whole(raw_drug_feats, wd2, bb[2:3])
    h = jnp.concatenate(
        [hg, hc, hd, jnp.zeros((NPAD - N, D), _f32)], axis=0)

    ei = edge_index.astype(jnp.int32)
    pad = jnp.full((EP - E,), N, dtype=jnp.int32)
    row16 = jnp.concatenate([ei[0], pad]).reshape(NS, CPT2, CHUNK)
    rowp = jnp.concatenate(
        [2 * row16, 2 * row16 + 1]).reshape(NC * NS, CPT2, CHUNK)
    col_flat = jnp.concatenate([ei[1], pad])
    colp_deg = col_flat.reshape(NW, CPT, CHUNK)
    colp = col_flat.reshape(NS, CPT2, CHUNK)

    ones16 = jnp.ones((CHUNK, 16), _f32)
    z16 = jnp.zeros((RPT, 16), _f32)
    z64 = jnp.zeros((RPT, F2), _f32)

    hist = _deg(colp_deg, ones16, z16)
    hist3 = hist.reshape(NC, NPAD, 16)
    y = _scale(h, hist3)
    y2 = y.reshape(NC * NPAD, F2)
    acc = _edge(y2, rowp, colp, z64)
    out = _final(acc, y, hist3, b_conv.reshape(1, D))
    return out


# speedup vs baseline: 1.3256x; 1.3256x over previous
import functools

import jax
import jax.numpy as jnp
from jax import lax
from jax.experimental import pallas as pl
from jax.experimental.pallas import tpu as pltpu
from jax.experimental.pallas import tpu_sc as plsc

N_GENE, N_CELL, N_DRUG = 8000, 1500, 500
N = N_GENE + N_CELL + N_DRUG
D = 128
F2 = D // 2
E = 320000

NC, NS = 2, 16
NW = NC * NS
CHUNK = 128
CPT = 80
CPT2 = 160
EP = NW * CPT * CHUNK
NPAD = 10112
RPT = NPAD // NS
NBUF = 4

_f32 = jnp.float32


def _fold_body(wg_ref, wc_ref, wd_ref, wcv_ref, bs_ref,
               wg2_ref, wc2_ref, wd2_ref, bb_ref):
    wcv = wcv_ref[...]
    wg2_ref[...] = jnp.dot(wg_ref[...], wcv, preferred_element_type=_f32)
    wc2_ref[...] = jnp.dot(wc_ref[...], wcv, preferred_element_type=_f32)
    wd2_ref[...] = jnp.dot(wd_ref[...], wcv, preferred_element_type=_f32)
    bb_ref[...] = jnp.dot(bs_ref[...], wcv, preferred_element_type=_f32)


def _fold(wg, wc, wd, wcv, bstack):
    return pl.pallas_call(
        _fold_body,
        out_shape=[
            jax.ShapeDtypeStruct((512, D), _f32),
            jax.ShapeDtypeStruct((256, D), _f32),
            jax.ShapeDtypeStruct((128, D), _f32),
            jax.ShapeDtypeStruct((3, D), _f32),
        ],
    )(wg, wc, wd, wcv, bstack)


def _proj_body(x_ref, w_ref, bb_ref, o_ref):
    o_ref[...] = jnp.dot(x_ref[...], w_ref[...],
                         preferred_element_type=_f32) + bb_ref[...]


def _proj_grid(x, w2, bb_row, blk):
    m, k = x.shape
    return pl.pallas_call(
        _proj_body,
        grid=(m // blk,),
        in_specs=[
            pl.BlockSpec((blk, k), lambda i: (i, 0)),
            pl.BlockSpec((k, D), lambda i: (0, 0)),
            pl.BlockSpec((1, D), lambda i: (0, 0)),
        ],
        out_specs=pl.BlockSpec((blk, D), lambda i: (i, 0)),
        out_shape=jax.ShapeDtypeStruct((m, D), _f32),
    )(x, w2, bb_row)


def _proj_whole(x, w2, bb_row):
    m = x.shape[0]
    return pl.pallas_call(
        _proj_body,
        out_shape=jax.ShapeDtypeStruct((m, D), _f32),
    )(x, w2, bb_row)


def _scale_body(h_ref, hist_ref, y2_ref):
    deg = 1.0 + hist_ref[0, :, 0:1] + hist_ref[1, :, 0:1]
    dinv = lax.rsqrt(deg)
    hv = h_ref[...] * dinv
    y2_ref[...] = jnp.concatenate([hv[:, :F2], hv[:, F2:]], axis=0)


def _scale(h, hist3):
    return pl.pallas_call(
        _scale_body,
        out_shape=jax.ShapeDtypeStruct((NC * NPAD, F2), _f32),
    )(h, hist3)


def _final_body(acc_ref, y2_ref, hist_ref, b_ref, o_ref):
    deg = 1.0 + hist_ref[0, :, 0:1] + hist_ref[1, :, 0:1]
    dinv = lax.rsqrt(deg)
    a = acc_ref[...] + y2_ref[...]
    o = jnp.concatenate([a[:N], a[NPAD:NPAD + N]], axis=1)
    o_ref[...] = o * dinv[:N] + b_ref[...]


def _final(acc, y2, hist3, brow):
    return pl.pallas_call(
        _final_body,
        out_shape=jax.ShapeDtypeStruct((N, D), _f32),
    )(acc, y2, hist3, brow)


_sc_mesh = plsc.VectorSubcoreMesh(
    core_axis_name="c", subcore_axis_name="s", num_cores=NC, num_subcores=NS)
_sc_params = pltpu.CompilerParams(use_tc_tiling_on_sc=False)


@functools.partial(
    pl.kernel,
    out_type=jax.ShapeDtypeStruct((NC * NPAD, 16), _f32),
    mesh=_sc_mesh,
    scratch_types=[
        pltpu.VMEM((CPT, CHUNK), jnp.int32),
        pltpu.VMEM((CHUNK, 16), _f32),
        pltpu.SemaphoreType.DMA,
        pltpu.VMEM_SHARED((NPAD, 16), _f32),
    ],
    compiler_params=_sc_params,
)
def _deg(colp_ref, ones_ref, z16_ref, out_ref, colbuf, ones_v, sem_sc, hist_sh):
    cid = lax.axis_index("c")
    sid = lax.axis_index("s")
    wid = sid * NC + cid
    pltpu.sync_copy(z16_ref, hist_sh.at[pl.ds(sid * RPT, RPT)])
    pltpu.sync_copy(ones_ref, ones_v)
    pltpu.sync_copy(colp_ref.at[wid], colbuf)
    plsc.subcore_barrier()

    def fire(j, carry):
        pltpu.make_async_copy(
            ones_v, hist_sh.at[colbuf.at[j]], sem_sc).start(add=True)
        return carry

    lax.fori_loop(0, CPT, fire, 0)

    def drain(j, carry):
        pltpu.make_async_copy(
            ones_v, hist_sh.at[colbuf.at[0]], sem_sc).wait()
        return carry

    lax.fori_loop(0, CPT, drain, 0)
    plsc.subcore_barrier()
    pltpu.sync_copy(hist_sh.at[pl.ds(sid * RPT, RPT)],
                    out_ref.at[pl.ds(cid * NPAD + sid * RPT, RPT)])


@functools.partial(
    pl.kernel,
    out_type=jax.ShapeDtypeStruct((NC * NPAD, F2), _f32),
    mesh=_sc_mesh,
    scratch_types=[
        pltpu.VMEM((CPT2, CHUNK), jnp.int32),
        pltpu.VMEM((CPT2, CHUNK), jnp.int32),
        pltpu.VMEM((NBUF, CHUNK, F2), _f32),
        pltpu.SemaphoreType.DMA,
        pltpu.SemaphoreType.DMA,
        pltpu.SemaphoreType.DMA,
        pltpu.SemaphoreType.DMA,
        pltpu.SemaphoreType.DMA,
        pltpu.SemaphoreType.DMA,
        pltpu.SemaphoreType.DMA,
        pltpu.SemaphoreType.DMA,
        pltpu.VMEM_SHARED((NPAD, F2), _f32),
    ],
    compiler_params=_sc_params,
)
def _edge(y2_ref, rowp_ref, colp_ref, z64_ref, out_ref,
          rowbuf, colbuf, rows, g0, g1, g2, g3, s0, s1, s2, s3, acc_sh):
    gsems = (g0, g1, g2, g3)
    ssems = (s0, s1, s2, s3)
    cid = lax.axis_index("c")
    sid = lax.axis_index("s")
    pltpu.sync_copy(z64_ref, acc_sh.at[pl.ds(sid * RPT, RPT)])
    pltpu.sync_copy(rowp_ref.at[cid * NS + sid], rowbuf)
    pltpu.sync_copy(colp_ref.at[sid], colbuf)
    plsc.subcore_barrier()

    for b in range(NBUF):
        pltpu.make_async_copy(
            y2_ref.at[rowbuf.at[b]], rows.at[b], gsems[b]).start()

    ngroups = CPT2 // NBUF

    def group(g, carry):
        for b in range(NBUF):
            j = g * NBUF + b
            pltpu.make_async_copy(
                y2_ref.at[rowbuf.at[b]], rows.at[b], gsems[b]).wait()
            pltpu.make_async_copy(
                rows.at[b], acc_sh.at[colbuf.at[j]], ssems[b]).start(add=True)
            pltpu.make_async_copy(
                rows.at[b], acc_sh.at[colbuf.at[0]], ssems[b]).wait()

            @pl.when(g < ngroups - 1)
            def _():
                pltpu.make_async_copy(
                    y2_ref.at[rowbuf.at[j + NBUF]], rows.at[b],
                    gsems[b]).start()
        return carry

    lax.fori_loop(0, ngroups, group, 0)
    plsc.subcore_barrier()
    pltpu.sync_copy(acc_sh.at[pl.ds(sid * RPT, RPT)],
                    out_ref.at[pl.ds(cid * NPAD + sid * RPT, RPT)])


def kernel(raw_gene_feats, raw_cell_feats, raw_drug_feats,
           gene_idx, cell_idx, drug_idx, edge_index,
           W_gene, b_gene, W_cell, b_cell, W_drug, b_drug, W_conv, b_conv):
    del gene_idx, cell_idx, drug_idx

    bstack = jnp.stack([b_gene, b_cell, b_drug])
    wg2, wc2, wd2, bb = _fold(W_gene, W_cell, W_drug, W_conv, bstack)

    hg = _proj_grid(raw_gene_feats, wg2, bb[0:1], 1000)
    hc = _proj_whole(raw_cell_feats, wc2, bb[1:2])
    hd = _proj_whole(raw_drug_feats, wd2, bb[2:3])
    h = jnp.concatenate(
        [hg, hc, hd, jnp.zeros((NPAD - N, D), _f32)], axis=0)

    ei = edge_index.astype(jnp.int32)
    pad = jnp.full((EP - E,), N, dtype=jnp.int32)
    row16 = jnp.concatenate([ei[0], pad]).reshape(NS, CPT2, CHUNK)
    rowp = jnp.concatenate([row16, row16 + NPAD]).reshape(NC * NS, CPT2, CHUNK)
    col_flat = jnp.concatenate([ei[1], pad])
    colp_deg = col_flat.reshape(NW, CPT, CHUNK)
    colp = col_flat.reshape(NS, CPT2, CHUNK)

    ones16 = jnp.ones((CHUNK, 16), _f32)
    z16 = jnp.zeros((RPT, 16), _f32)
    z64 = jnp.zeros((RPT, F2), _f32)

    hist = _deg(colp_deg, ones16, z16)
    hist3 = hist.reshape(NC, NPAD, 16)
    y2 = _scale(h, hist3)
    acc = _edge(y2, rowp, colp, z64)
    out = _final(acc, y2, hist3, b_conv.reshape(1, D))
    return out


# speedup vs baseline: 1.3288x; 1.0024x over previous
import functools

import jax
import jax.numpy as jnp
from jax import lax
from jax.experimental import pallas as pl
from jax.experimental.pallas import tpu as pltpu
from jax.experimental.pallas import tpu_sc as plsc

N_GENE, N_CELL, N_DRUG = 8000, 1500, 500
N = N_GENE + N_CELL + N_DRUG
D = 128
F2 = D // 2
E = 320000

NC, NS = 2, 16
NW = NC * NS
CHUNK = 128
CPT = 80
CPT2 = 160
EP = NW * CPT * CHUNK
NPAD = 10112
RPT = NPAD // NS
NBUF = 5

_f32 = jnp.float32


def _fold_body(wg_ref, wc_ref, wd_ref, wcv_ref, bs_ref,
               wg2_ref, wc2_ref, wd2_ref, bb_ref):
    wcv = wcv_ref[...]
    wg2_ref[...] = jnp.dot(wg_ref[...], wcv, preferred_element_type=_f32)
    wc2_ref[...] = jnp.dot(wc_ref[...], wcv, preferred_element_type=_f32)
    wd2_ref[...] = jnp.dot(wd_ref[...], wcv, preferred_element_type=_f32)
    bb_ref[...] = jnp.dot(bs_ref[...], wcv, preferred_element_type=_f32)


def _fold(wg, wc, wd, wcv, bstack):
    return pl.pallas_call(
        _fold_body,
        out_shape=[
            jax.ShapeDtypeStruct((512, D), _f32),
            jax.ShapeDtypeStruct((256, D), _f32),
            jax.ShapeDtypeStruct((128, D), _f32),
            jax.ShapeDtypeStruct((3, D), _f32),
        ],
    )(wg, wc, wd, wcv, bstack)


def _proj_body(x_ref, w_ref, bb_ref, o_ref):
    o_ref[...] = jnp.dot(x_ref[...], w_ref[...],
                         preferred_element_type=_f32) + bb_ref[...]


def _proj_grid(x, w2, bb_row, blk):
    m, k = x.shape
    return pl.pallas_call(
        _proj_body,
        grid=(m // blk,),
        in_specs=[
            pl.BlockSpec((blk, k), lambda i: (i, 0)),
            pl.BlockSpec((k, D), lambda i: (0, 0)),
            pl.BlockSpec((1, D), lambda i: (0, 0)),
        ],
        out_specs=pl.BlockSpec((blk, D), lambda i: (i, 0)),
        out_shape=jax.ShapeDtypeStruct((m, D), _f32),
    )(x, w2, bb_row)


def _proj_whole(x, w2, bb_row):
    m = x.shape[0]
    return pl.pallas_call(
        _proj_body,
        out_shape=jax.ShapeDtypeStruct((m, D), _f32),
    )(x, w2, bb_row)


def _scale_body(h_ref, hist_ref, y2_ref):
    deg = 1.0 + hist_ref[0, :, 0:1] + hist_ref[1, :, 0:1]
    dinv = lax.rsqrt(deg)
    hv = h_ref[...] * dinv
    y2_ref[...] = jnp.concatenate([hv[:, :F2], hv[:, F2:]], axis=0)


def _scale(h, hist3):
    return pl.pallas_call(
        _scale_body,
        out_shape=jax.ShapeDtypeStruct((NC * NPAD, F2), _f32),
    )(h, hist3)


def _final_body(acc_ref, y2_ref, hist_ref, b_ref, o_ref):
    deg = 1.0 + hist_ref[0, :, 0:1] + hist_ref[1, :, 0:1]
    dinv = lax.rsqrt(deg)
    a = acc_ref[...] + y2_ref[...]
    o = jnp.concatenate([a[:N], a[NPAD:NPAD + N]], axis=1)
    o_ref[...] = o * dinv[:N] + b_ref[...]


def _final(acc, y2, hist3, brow):
    return pl.pallas_call(
        _final_body,
        out_shape=jax.ShapeDtypeStruct((N, D), _f32),
    )(acc, y2, hist3, brow)


_sc_mesh = plsc.VectorSubcoreMesh(
    core_axis_name="c", subcore_axis_name="s", num_cores=NC, num_subcores=NS)
_sc_params = pltpu.CompilerParams(use_tc_tiling_on_sc=False)


@functools.partial(
    pl.kernel,
    out_type=jax.ShapeDtypeStruct((NC * NPAD, 16), _f32),
    mesh=_sc_mesh,
    scratch_types=[
        pltpu.VMEM((CPT, CHUNK), jnp.int32),
        pltpu.VMEM((CHUNK, 16), _f32),
        pltpu.SemaphoreType.DMA,
        pltpu.VMEM_SHARED((NPAD, 16), _f32),
    ],
    compiler_params=_sc_params,
)
def _deg(colp_ref, ones_ref, z16_ref, out_ref, colbuf, ones_v, sem_sc, hist_sh):
    cid = lax.axis_index("c")
    sid = lax.axis_index("s")
    wid = sid * NC + cid
    pltpu.sync_copy(z16_ref, hist_sh.at[pl.ds(sid * RPT, RPT)])
    pltpu.sync_copy(ones_ref, ones_v)
    pltpu.sync_copy(colp_ref.at[wid], colbuf)
    plsc.subcore_barrier()

    def fire(j, carry):
        pltpu.make_async_copy(
            ones_v, hist_sh.at[colbuf.at[j]], sem_sc).start(add=True)
        return carry

    lax.fori_loop(0, CPT, fire, 0)

    def drain(j, carry):
        pltpu.make_async_copy(
            ones_v, hist_sh.at[colbuf.at[0]], sem_sc).wait()
        return carry

    lax.fori_loop(0, CPT, drain, 0)
    plsc.subcore_barrier()
    pltpu.sync_copy(hist_sh.at[pl.ds(sid * RPT, RPT)],
                    out_ref.at[pl.ds(cid * NPAD + sid * RPT, RPT)])


@functools.partial(
    pl.kernel,
    out_type=jax.ShapeDtypeStruct((NC * NPAD, F2), _f32),
    mesh=_sc_mesh,
    scratch_types=[
        pltpu.VMEM((CPT2, CHUNK), jnp.int32),
        pltpu.VMEM((CPT2, CHUNK), jnp.int32),
        pltpu.VMEM((NBUF, CHUNK, F2), _f32),
        pltpu.SemaphoreType.DMA,
        pltpu.SemaphoreType.DMA,
        pltpu.SemaphoreType.DMA,
        pltpu.SemaphoreType.DMA,
        pltpu.SemaphoreType.DMA,
        pltpu.SemaphoreType.DMA,
        pltpu.SemaphoreType.DMA,
        pltpu.SemaphoreType.DMA,
        pltpu.SemaphoreType.DMA,
        pltpu.SemaphoreType.DMA,
        pltpu.VMEM_SHARED((NPAD, F2), _f32),
    ],
    compiler_params=_sc_params,
)
def _edge(y2_ref, rowp_ref, colp_ref, z64_ref, out_ref,
          rowbuf, colbuf, rows, g0, g1, g2, g3, g4,
          s0, s1, s2, s3, s4, acc_sh):
    gsems = (g0, g1, g2, g3, g4)
    ssems = (s0, s1, s2, s3, s4)
    cid = lax.axis_index("c")
    sid = lax.axis_index("s")
    pltpu.sync_copy(z64_ref, acc_sh.at[pl.ds(sid * RPT, RPT)])
    pltpu.sync_copy(rowp_ref.at[cid * NS + sid], rowbuf)
    pltpu.sync_copy(colp_ref.at[sid], colbuf)
    plsc.subcore_barrier()

    for b in range(NBUF):
        pltpu.make_async_copy(
            y2_ref.at[rowbuf.at[b]], rows.at[b], gsems[b]).start()

    ngroups = CPT2 // NBUF

    def group(g, carry):
        for b in range(NBUF):
            j = g * NBUF + b
            pltpu.make_async_copy(
                y2_ref.at[rowbuf.at[b]], rows.at[b], gsems[b]).wait()
            pltpu.make_async_copy(
                rows.at[b], acc_sh.at[colbuf.at[j]], ssems[b]).start(add=True)
            pltpu.make_async_copy(
                rows.at[b], acc_sh.at[colbuf.at[0]], ssems[b]).wait()

            @pl.when(g < ngroups - 1)
            def _():
                pltpu.make_async_copy(
                    y2_ref.at[rowbuf.at[j + NBUF]], rows.at[b],
                    gsems[b]).start()
        return carry

    lax.fori_loop(0, ngroups, group, 0)
    plsc.subcore_barrier()
    pltpu.sync_copy(acc_sh.at[pl.ds(sid * RPT, RPT)],
                    out_ref.at[pl.ds(cid * NPAD + sid * RPT, RPT)])


def kernel(raw_gene_feats, raw_cell_feats, raw_drug_feats,
           gene_idx, cell_idx, drug_idx, edge_index,
           W_gene, b_gene, W_cell, b_cell, W_drug, b_drug, W_conv, b_conv):
    del gene_idx, cell_idx, drug_idx

    bstack = jnp.stack([b_gene, b_cell, b_drug])
    wg2, wc2, wd2, bb = _fold(W_gene, W_cell, W_drug, W_conv, bstack)

    hg = _proj_grid(raw_gene_feats, wg2, bb[0:1], 1000)
    hc = _proj_whole(raw_cell_feats, wc2, bb[1:2])
    hd = _proj_whole(raw_drug_feats, wd2, bb[2:3])
    h = jnp.concatenate(
        [hg, hc, hd, jnp.zeros((NPAD - N, D), _f32)], axis=0)

    ei = edge_index.astype(jnp.int32)
    pad = jnp.full((EP - E,), N, dtype=jnp.int32)
    row16 = jnp.concatenate([ei[0], pad]).reshape(NS, CPT2, CHUNK)
    rowp = jnp.concatenate([row16, row16 + NPAD]).reshape(NC * NS, CPT2, CHUNK)
    col_flat = jnp.concatenate([ei[1], pad])
    colp_deg = col_flat.reshape(NW, CPT, CHUNK)
    colp = col_flat.reshape(NS, CPT2, CHUNK)

    ones16 = jnp.ones((CHUNK, 16), _f32)
    z16 = jnp.zeros((RPT, 16), _f32)
    z64 = jnp.zeros((RPT, F2), _f32)

    hist = _deg(colp_deg, ones16, z16)
    hist3 = hist.reshape(NC, NPAD, 16)
    y2 = _scale(h, hist3)
    acc = _edge(y2, rowp, colp, z64)
    out = _final(acc, y2, hist3, b_conv.reshape(1, D))
    return out
